# Initial kernel scaffold; baseline (speedup 1.0000x reference)
import jax
import jax.numpy as jnp
from jax.experimental import pallas as pl


def kernel(inputs, indices):
    raise NotImplementedError("write your pallas kernel here")


# speedup vs baseline: 1.1870x; 1.1870x over previous
import jax
import jax.numpy as jnp
from jax import lax
from jax.experimental import pallas as pl
from jax.experimental.pallas import tpu as pltpu
from jax.experimental.pallas import tpu_sc as plsc

INPUT_LEN = 1000000
NUM_OUTPUTS = 409600
NUM_WORKERS = 32
PER_TILE = NUM_OUTPUTS // NUM_WORKERS
LANES = 16
NVEC = PER_TILE // LANES


def _sc_body(inp_hbm, idx_hbm, out_hbm, idx_v, vals_v, out_v, sem):
    wid = lax.axis_index("s") * 2 + lax.axis_index("c")
    base = wid * PER_TILE
    pltpu.sync_copy(idx_hbm.at[pl.ds(base, PER_TILE)], idx_v)
    pltpu.async_copy(inp_hbm.at[idx_v], vals_v, sem).wait()

    @pl.loop(0, NVEC)
    def thresh(i):
        v = vals_v[pl.ds(i * LANES, LANES)]
        out_v[pl.ds(i * LANES, LANES)] = jnp.where(
            v >= 64, jnp.int32(1), jnp.int32(0)
        )

    pltpu.sync_copy(out_v, out_hbm.at[pl.ds(base, PER_TILE)])


@jax.jit
def kernel(inputs, indices):
    mesh = plsc.VectorSubcoreMesh(core_axis_name="c", subcore_axis_name="s")
    call = pl.kernel(
        _sc_body,
        out_type=jax.ShapeDtypeStruct((NUM_OUTPUTS,), jnp.int32),
        mesh=mesh,
        scratch_types=[
            pltpu.VMEM((PER_TILE,), jnp.int32),
            pltpu.VMEM((PER_TILE,), jnp.int32),
            pltpu.VMEM((PER_TILE,), jnp.int32),
            pltpu.SemaphoreType.DMA,
        ],
    )
    return call(inputs, indices).astype(jnp.bool_)


# speedup vs baseline: 1.2578x; 1.0597x over previous
import jax
import jax.numpy as jnp
from jax import lax
from jax.experimental import pallas as pl
from jax.experimental.pallas import tpu as pltpu
from jax.experimental.pallas import tpu_sc as plsc

INPUT_LEN = 1000000
NUM_OUTPUTS = 409600
NUM_WORKERS = 32
PER_TILE = NUM_OUTPUTS // NUM_WORKERS
LANES = 16
NVEC = PER_TILE // LANES

BLOCK = 512
WPB = BLOCK // 32
NBLK_FULL = INPUT_LEN // BLOCK
TAIL_VALS = INPUT_LEN - NBLK_FULL * BLOCK
NBLK = NBLK_FULL + 1
PACKED_WORDS = NBLK * WPB
BLK_PER_TILE = 123
BLK_LAST = NBLK_FULL - 15 * BLK_PER_TILE
VALS_PER_TILE = BLK_PER_TILE * BLOCK
VALS_LAST = BLK_LAST * BLOCK + TAIL_VALS


def _pack_block(stage_v, packed_v, local_blk, nbits):
    acc = jnp.zeros((LANES,), jnp.int32)
    for b in range(nbits):
        v = stage_v[pl.ds(local_blk * BLOCK + b * LANES, LANES)]
        w = (1 << b) if b < 31 else -(1 << 31)
        acc = acc | jnp.where(v >= 64, jnp.int32(w), jnp.int32(0))
    packed_v[pl.ds(local_blk * WPB, WPB)] = acc


def _sc_body(inp_hbm, idx_hbm, out_hbm, scratch_hbm,
             stage_v, packed_v, idx_v, out_v):
    c = lax.axis_index("c")
    s = lax.axis_index("s")
    wid = s * 2 + c

    val0 = s * VALS_PER_TILE

    @pl.when(s < 15)
    def _():
        pltpu.sync_copy(inp_hbm.at[pl.ds(val0, VALS_PER_TILE)],
                        stage_v.at[pl.ds(0, VALS_PER_TILE)])

        @pl.loop(0, BLK_PER_TILE)
        def _(g):
            _pack_block(stage_v, packed_v, g, 32)

    @pl.when(s == 15)
    def _():
        pltpu.sync_copy(inp_hbm.at[pl.ds(val0, VALS_LAST)],
                        stage_v.at[pl.ds(0, VALS_LAST)])

        @pl.loop(0, BLK_LAST)
        def _(g):
            _pack_block(stage_v, packed_v, g, 32)

        _pack_block(stage_v, packed_v, BLK_LAST, TAIL_VALS // LANES)

    word0 = c * PACKED_WORDS + s * (BLK_PER_TILE * WPB)

    @pl.when(s < 15)
    def _():
        pltpu.sync_copy(packed_v.at[pl.ds(0, BLK_PER_TILE * WPB)],
                        scratch_hbm.at[pl.ds(word0, BLK_PER_TILE * WPB)])

    @pl.when(s == 15)
    def _():
        pltpu.sync_copy(packed_v.at[pl.ds(0, (BLK_LAST + 1) * WPB)],
                        scratch_hbm.at[pl.ds(word0, (BLK_LAST + 1) * WPB)])

    plsc.subcore_barrier()
    pltpu.sync_copy(scratch_hbm.at[pl.ds(c * PACKED_WORDS, PACKED_WORDS)],
                    packed_v)

    base = wid * PER_TILE
    pltpu.sync_copy(idx_hbm.at[pl.ds(base, PER_TILE)], idx_v)

    @pl.loop(0, NVEC)
    def _(i):
        iv = idx_v[pl.ds(i * LANES, LANES)]
        wordpos = ((iv >> 9) << 4) | (iv & 15)
        bit = (iv >> 4) & 31
        w = plsc.load_gather(packed_v, [wordpos])
        out_v[pl.ds(i * LANES, LANES)] = (w >> bit) & 1

    pltpu.sync_copy(out_v, out_hbm.at[pl.ds(base, PER_TILE)])


@jax.jit
def kernel(inputs, indices):
    mesh = plsc.VectorSubcoreMesh(core_axis_name="c", subcore_axis_name="s")
    call = pl.kernel(
        _sc_body,
        out_type=(
            jax.ShapeDtypeStruct((NUM_OUTPUTS,), jnp.int32),
            jax.ShapeDtypeStruct((2 * PACKED_WORDS,), jnp.int32),
        ),
        mesh=mesh,
        scratch_types=[
            pltpu.VMEM((VALS_PER_TILE,), jnp.int32),
            pltpu.VMEM((PACKED_WORDS,), jnp.int32),
            pltpu.VMEM((PER_TILE,), jnp.int32),
            pltpu.VMEM((PER_TILE,), jnp.int32),
        ],
        compiler_params=pltpu.CompilerParams(needs_layout_passes=False),
    )
    out_i32, _ = call(inputs, indices)
    return out_i32.astype(jnp.bool_)
